# Initial kernel scaffold; baseline (speedup 1.0000x reference)
import functools

import jax
import jax.numpy as jnp
from jax import lax
from jax.experimental import pallas as pl
from jax.experimental.pallas import tpu as pltpu
from jax.experimental.pallas import tpu_sc as plsc

N = 10000
E = 320000
D = 128
C = 8
H = 16
F = C * H

NC = 2
NS = 16
NW = NC * NS
EDGES_PER_W = E // NW
CHUNK = 80
NCHUNK = EDGES_PER_W // CHUNK
ROWS_PER_SUB = N // NS
ZROWS = 125


def _sc_agg_body(h_hbm, src_hbm, dst_hbm, om_hbm, out_hbm,
                 src_v, dst_v, om_v, rows_v, zbuf_v, acc_sh, sem):
    cid = lax.axis_index("c")
    sid = lax.axis_index("s")
    wid = cid * NS + sid

    def zbody(i, carry):
        r = i // C
        g = i - r * C
        zbuf_v[r, pl.ds(g * H, H)] = jnp.zeros((H,), jnp.float32)
        return carry
    lax.fori_loop(0, ZROWS * C, zbody, 0)
    for j in range(ROWS_PER_SUB // ZROWS):
        pltpu.sync_copy(zbuf_v, acc_sh.at[pl.ds(sid * ROWS_PER_SUB + j * ZROWS, ZROWS)])
    plsc.subcore_barrier()

    ebase = wid * EDGES_PER_W

    def chunk_body(t, carry):
        base = ebase + t * CHUNK
        pltpu.sync_copy(src_hbm.at[pl.ds(base, CHUNK)], src_v)
        pltpu.sync_copy(dst_hbm.at[pl.ds(base, CHUNK)], dst_v)
        pltpu.sync_copy(om_hbm.at[pl.ds(base * C, CHUNK * C)], om_v)
        pltpu.async_copy(h_hbm.at[src_v], rows_v, sem).wait()

        def edge_body(e, ecarry):
            for c in range(C):
                idx = jnp.full((H,), e * C + c, jnp.int32)
                om = plsc.load_gather(om_v, [idx])
                rows_v[e, pl.ds(c * H, H)] = rows_v[e, pl.ds(c * H, H)] * om
            return ecarry
        lax.fori_loop(0, CHUNK, edge_body, 0)

        pltpu.sync_copy(rows_v, acc_sh.at[dst_v], add=True)
        return carry

    lax.fori_loop(0, NCHUNK, chunk_body, 0)
    plsc.subcore_barrier()

    pltpu.sync_copy(acc_sh.at[pl.ds(sid * ROWS_PER_SUB, ROWS_PER_SUB)],
                    out_hbm.at[cid, pl.ds(sid * ROWS_PER_SUB, ROWS_PER_SUB)])


_sc_agg = functools.partial(
    pl.kernel,
    out_type=jax.ShapeDtypeStruct((NC, N, F), jnp.float32),
    mesh=plsc.VectorSubcoreMesh(core_axis_name="c", subcore_axis_name="s",
                                num_cores=NC, num_subcores=NS),
    scratch_types=[
        pltpu.VMEM((CHUNK,), jnp.int32),
        pltpu.VMEM((CHUNK,), jnp.int32),
        pltpu.VMEM((CHUNK * C,), jnp.float32),
        pltpu.VMEM((CHUNK, F), jnp.float32),
        pltpu.VMEM((ZROWS, F), jnp.float32),
        pltpu.VMEM_SHARED((N, F), jnp.float32),
        pltpu.SemaphoreType.DMA,
    ],
)(_sc_agg_body)


_BN = 1250


def _proj_body(x_ref, p_ref, o_ref):
    o_ref[...] = jnp.dot(x_ref[...], p_ref[...], preferred_element_type=jnp.float32)


def _post_body(parts_ref, wbd_ref, mavg_ref, g_ref, b_ref, o_ref, *, do_relu):
    s = parts_ref[0] + parts_ref[1]
    t = jnp.dot(s, wbd_ref[...], preferred_element_type=jnp.float32)
    mu = jnp.dot(t, mavg_ref[...], preferred_element_type=jnp.float32)
    d = t - mu
    var = jnp.dot(d * d, mavg_ref[...], preferred_element_type=jnp.float32)
    y = g_ref[...] * d * lax.rsqrt(var + 1e-5) + b_ref[...]
    if do_relu:
        y = jnp.maximum(y, 0.0)
    o_ref[...] = y


def _tc_proj(x, pflat):
    return pl.pallas_call(
        _proj_body,
        out_shape=jax.ShapeDtypeStruct((N, F), jnp.float32),
        grid=(N // _BN,),
        in_specs=[pl.BlockSpec((_BN, D), lambda i: (i, 0)),
                  pl.BlockSpec((D, F), lambda i: (0, 0))],
        out_specs=pl.BlockSpec((_BN, F), lambda i: (i, 0)),
    )(x, pflat)


def _tc_post(parts, wbd, mavg, gamma_t, beta_t, do_relu):
    return pl.pallas_call(
        functools.partial(_post_body, do_relu=do_relu),
        out_shape=jax.ShapeDtypeStruct((N, F), jnp.float32),
        grid=(N // _BN,),
        in_specs=[pl.BlockSpec((NC, _BN, F), lambda i: (0, i, 0)),
                  pl.BlockSpec((F, F), lambda i: (0, 0)),
                  pl.BlockSpec((F, F), lambda i: (0, 0)),
                  pl.BlockSpec((1, F), lambda i: (0, 0)),
                  pl.BlockSpec((1, F), lambda i: (0, 0))],
        out_specs=pl.BlockSpec((_BN, F), lambda i: (i, 0)),
    )(parts, wbd, mavg, gamma_t, beta_t)


def _blockdiag(w):
    eye = jnp.eye(C, dtype=w.dtype)
    return jnp.einsum('chk,cd->chdk', w, eye).reshape(C * H, C * w.shape[-1])


def kernel(x, edge_index, omega, proj0, w0, proj1, w1, ln_gamma, ln_beta):
    src = edge_index[0]
    dst = edge_index[1]
    om_flat = omega.reshape(E * C)

    mavg = jnp.kron(jnp.eye(C, dtype=jnp.float32),
                    jnp.full((H, H), 1.0 / H, dtype=jnp.float32))
    gamma_t = jnp.tile(ln_gamma, C).reshape(1, F)
    beta_t = jnp.tile(ln_beta, C).reshape(1, F)

    h0 = _tc_proj(x, proj0.reshape(D, F))
    parts1 = _sc_agg(h0, src, dst, om_flat)
    h1 = _tc_post(parts1, _blockdiag(w0), mavg, gamma_t, beta_t, True)
    parts2 = _sc_agg(h1, src, dst, om_flat)
    h2 = _tc_post(parts2, _blockdiag(w1), mavg, gamma_t, beta_t, False)
    return h2.reshape(N, C, H)


# speedup vs baseline: 68.8690x; 68.8690x over previous
import functools

import jax
import jax.numpy as jnp
from jax import lax
from jax.experimental import pallas as pl
from jax.experimental.pallas import tpu as pltpu
from jax.experimental.pallas import tpu_sc as plsc

N = 10000
E = 320000
D = 128
C = 8
H = 16
F = C * H

NC = 2
NS = 16
NW = NC * NS
EDGES_PER_W = E // NW
CHUNK = 80
NCHUNK = EDGES_PER_W // CHUNK
ROW_STRIDE = 624
ROW_SPAN = 640
ZROWS = 128


def _sc_agg_body(h_hbm, src_hbm, dst_hbm, om_hbm, out_hbm,
                 src_v, dst_v, om_v, rows_v, zbuf_v, acc_sh, sem):
    cid = lax.axis_index("c")
    sid = lax.axis_index("s")
    wid = cid * NS + sid

    def zbody(i, carry):
        r = i // C
        g = i - r * C
        zbuf_v[r, pl.ds(g * H, H)] = jnp.zeros((H,), jnp.float32)
        return carry
    lax.fori_loop(0, ZROWS * C, zbody, 0)
    for j in range(ROW_SPAN // ZROWS):
        pltpu.sync_copy(zbuf_v, acc_sh.at[pl.ds(sid * ROW_STRIDE + j * ZROWS, ZROWS)])
    plsc.subcore_barrier()

    ebase = wid * EDGES_PER_W

    def chunk_body(t, carry):
        base = ebase + t * CHUNK
        pltpu.sync_copy(src_hbm.at[pl.ds(base, CHUNK)], src_v)
        pltpu.sync_copy(dst_hbm.at[pl.ds(base, CHUNK)], dst_v)
        pltpu.sync_copy(om_hbm.at[pl.ds(base * C, CHUNK * C)], om_v)
        pltpu.async_copy(h_hbm.at[src_v], rows_v, sem).wait()

        def pair_body(t, ecarry):
            om16 = om_v[pl.ds(t * 2 * C, 2 * C)]
            e0 = t * 2
            for j in range(2 * C):
                e = e0 + j // C
                sl = pl.ds((j % C) * H, H)
                rows_v[e, sl] = rows_v[e, sl] * om16[j]
            return ecarry
        lax.fori_loop(0, CHUNK // 2, pair_body, 0)

        pltpu.sync_copy(rows_v, acc_sh.at[dst_v], add=True)
        return carry

    lax.fori_loop(0, NCHUNK, chunk_body, 0)
    plsc.subcore_barrier()

    for j in range(ROW_SPAN // ZROWS):
        r0 = sid * ROW_STRIDE + j * ZROWS
        pltpu.sync_copy(acc_sh.at[pl.ds(r0, ZROWS)],
                        out_hbm.at[cid, pl.ds(r0, ZROWS)])


_sc_agg = functools.partial(
    pl.kernel,
    out_type=jax.ShapeDtypeStruct((NC, N, F), jnp.float32),
    mesh=plsc.VectorSubcoreMesh(core_axis_name="c", subcore_axis_name="s",
                                num_cores=NC, num_subcores=NS),
    scratch_types=[
        pltpu.VMEM((CHUNK,), jnp.int32),
        pltpu.VMEM((CHUNK,), jnp.int32),
        pltpu.VMEM((CHUNK * C,), jnp.float32),
        pltpu.VMEM((CHUNK, F), jnp.float32),
        pltpu.VMEM((ZROWS, F), jnp.float32),
        pltpu.VMEM_SHARED((N, F), jnp.float32),
        pltpu.SemaphoreType.DMA,
    ],
)(_sc_agg_body)


_BN = 1000


def _proj_body(x_ref, p_ref, o_ref):
    o_ref[...] = jnp.dot(x_ref[...], p_ref[...], preferred_element_type=jnp.float32)


def _post_body(parts_ref, wbd_ref, mavg_ref, g_ref, b_ref, o_ref, *, do_relu):
    s = parts_ref[0] + parts_ref[1]
    t = jnp.dot(s, wbd_ref[...], preferred_element_type=jnp.float32)
    mu = jnp.dot(t, mavg_ref[...], preferred_element_type=jnp.float32)
    d = t - mu
    var = jnp.dot(d * d, mavg_ref[...], preferred_element_type=jnp.float32)
    y = g_ref[...] * d * lax.rsqrt(var + 1e-5) + b_ref[...]
    if do_relu:
        y = jnp.maximum(y, 0.0)
    o_ref[...] = y


def _tc_proj(x, pflat):
    return pl.pallas_call(
        _proj_body,
        out_shape=jax.ShapeDtypeStruct((N, F), jnp.float32),
        grid=(N // _BN,),
        in_specs=[pl.BlockSpec((_BN, D), lambda i: (i, 0)),
                  pl.BlockSpec((D, F), lambda i: (0, 0))],
        out_specs=pl.BlockSpec((_BN, F), lambda i: (i, 0)),
    )(x, pflat)


def _tc_post(parts, wbd, mavg, gamma_t, beta_t, do_relu):
    return pl.pallas_call(
        functools.partial(_post_body, do_relu=do_relu),
        out_shape=jax.ShapeDtypeStruct((N, F), jnp.float32),
        grid=(N // _BN,),
        in_specs=[pl.BlockSpec((NC, _BN, F), lambda i: (0, i, 0)),
                  pl.BlockSpec((F, F), lambda i: (0, 0)),
                  pl.BlockSpec((F, F), lambda i: (0, 0)),
                  pl.BlockSpec((1, F), lambda i: (0, 0)),
                  pl.BlockSpec((1, F), lambda i: (0, 0))],
        out_specs=pl.BlockSpec((_BN, F), lambda i: (i, 0)),
    )(parts, wbd, mavg, gamma_t, beta_t)


def _blockdiag(w):
    eye = jnp.eye(C, dtype=w.dtype)
    return jnp.einsum('chk,cd->chdk', w, eye).reshape(C * H, C * w.shape[-1])


def kernel(x, edge_index, omega, proj0, w0, proj1, w1, ln_gamma, ln_beta):
    src = edge_index[0]
    dst = edge_index[1]
    om_flat = omega.reshape(E * C)

    mavg = jnp.kron(jnp.eye(C, dtype=jnp.float32),
                    jnp.full((H, H), 1.0 / H, dtype=jnp.float32))
    gamma_t = jnp.tile(ln_gamma, C).reshape(1, F)
    beta_t = jnp.tile(ln_beta, C).reshape(1, F)

    h0 = _tc_proj(x, proj0.reshape(D, F))
    parts1 = _sc_agg(h0, src, dst, om_flat)
    h1 = _tc_post(parts1, _blockdiag(w0), mavg, gamma_t, beta_t, True)
    parts2 = _sc_agg(h1, src, dst, om_flat)
    h2 = _tc_post(parts2, _blockdiag(w1), mavg, gamma_t, beta_t, False)
    return h2.reshape(N, C, H)
